# baseline (speedup 1.0000x reference)
import jax
import jax.numpy as jnp
from jax import lax
from jax.experimental import pallas as pl

_NUM_PHASES = 8
_BLK = 8192


def _body(obs_ref, ph_ref, out_ref):
    blk, obs_w = obs_ref.shape
    out_ref[:, :obs_w] = obs_ref[...]
    ph = ph_ref[...]
    cols = lax.broadcasted_iota(jnp.int32, (blk, _NUM_PHASES), 1)
    out_ref[:, obs_w:] = (cols == ph).astype(jnp.float32)


def kernel(obs, phases):
    rows, obs_w = obs.shape
    ph2 = jnp.broadcast_to(
        phases.astype(jnp.int32)[:, None], (rows, _NUM_PHASES)
    )
    return pl.pallas_call(
        _body,
        grid=(rows // _BLK,),
        in_specs=[
            pl.BlockSpec((_BLK, obs_w), lambda i: (i, 0)),
            pl.BlockSpec((_BLK, _NUM_PHASES), lambda i: (i, 0)),
        ],
        out_specs=pl.BlockSpec((_BLK, obs_w + _NUM_PHASES), lambda i: (i, 0)),
        out_shape=jax.ShapeDtypeStruct((rows, obs_w + _NUM_PHASES), jnp.float32),
    )(obs, ph2)


# speedup vs baseline: 1.5534x; 1.5534x over previous
import jax
import jax.numpy as jnp
from jax import lax
from jax.experimental import pallas as pl

_NUM_PHASES = 8
_BLK = 8192


def _body(obs_ref, ph_ref, out_ref):
    blk, obs_w = obs_ref.shape
    out_ref[:, :obs_w] = obs_ref[...]
    ph = ph_ref[...]
    rows_iota = lax.broadcasted_iota(jnp.int32, (_NUM_PHASES, blk), 0)
    tail_t = (rows_iota == ph[None, :]).astype(jnp.float32)
    out_ref[:, obs_w:] = tail_t.T


def kernel(obs, phases):
    rows, obs_w = obs.shape
    return pl.pallas_call(
        _body,
        grid=(rows // _BLK,),
        in_specs=[
            pl.BlockSpec((_BLK, obs_w), lambda i: (i, 0)),
            pl.BlockSpec((_BLK,), lambda i: (i,)),
        ],
        out_specs=pl.BlockSpec((_BLK, obs_w + _NUM_PHASES), lambda i: (i, 0)),
        out_shape=jax.ShapeDtypeStruct((rows, obs_w + _NUM_PHASES), jnp.float32),
    )(obs, phases.astype(jnp.int32))
